# baseline (speedup 1.0000x reference)
import functools

import jax
import jax.numpy as jnp
from jax import lax
from jax.experimental import pallas as pl
from jax.experimental.pallas import tpu as pltpu
from jax.experimental.pallas import tpu_sc as plsc

D = 64
B = 16
L = 2048
N = B * L
NC = 2
NS = 16
LANES = 16
ROWS = D // NC
G = L // LANES
RPP = 8
NPASS = ROWS // RPP


def _body(p_hbm, species_hbm, out_hbm,
          species_v, rank_v, in_b0, in_b1, out_b0, out_b1, cnt_v, run_v,
          sem_i0, sem_i1, sem_o0, sem_o1):
    c = lax.axis_index("c")
    s = lax.axis_index("s")
    r0 = c * ROWS
    col0 = s * L

    in_bufs = (in_b0, in_b1)
    out_bufs = (out_b0, out_b1)
    in_sems = (sem_i0, sem_i1)
    out_sems = (sem_o0, sem_o1)

    def start_in(k):
        cp = pltpu.make_async_copy(
            p_hbm.at[pl.ds(r0 + k * RPP, RPP), pl.ds(col0, L)],
            in_bufs[k % 2], in_sems[k % 2])
        cp.start()
        return cp

    in_pending = [start_in(0)]

    pltpu.sync_copy(species_hbm.at[s], species_v)

    iota = lax.iota(jnp.int32, LANES)
    zeros = jnp.zeros((LANES,), jnp.int32)
    cnt_v[pl.ds(0, LANES)] = zeros

    def cnt_body(g, _):
        svec = species_v[pl.ds(g * LANES, LANES)]
        sc, last = plsc.scan_count(svec)
        tgt = jnp.where(last, svec, LANES + iota)
        plsc.addupdate_scatter(cnt_v, [tgt], sc)
        return 0

    lax.fori_loop(0, G, cnt_body, 0)

    cnt = cnt_v[pl.ds(0, LANES)]
    off = L - plsc.cumsum(cnt)
    run_v[pl.ds(0, LANES)] = off

    def rank_body(g, _):
        svec = species_v[pl.ds(g * LANES, LANES)]
        sc, last = plsc.scan_count(svec)
        base = plsc.load_gather(run_v, [svec])
        rank_v[pl.ds(g * LANES, LANES)] = jnp.clip(base + sc - 1, 0, L - 1)
        tgt = jnp.where(last, svec, LANES + iota)
        plsc.addupdate_scatter(run_v, [tgt], sc)
        return 0

    lax.fori_loop(0, G, rank_body, 0)

    out_pending = [None, None]
    for k in range(NPASS):
        slot = k % 2
        in_pending[k].wait()
        if k + 1 < NPASS:
            in_pending.append(start_in(k + 1))
        if out_pending[slot] is not None:
            out_pending[slot].wait()
        ibuf = in_bufs[slot]
        obuf = out_bufs[slot]

        def make_pass(ibuf, obuf):
            def pass_fn(g):
                pos = rank_v[pl.ds(g * LANES, LANES)]
                for r in range(RPP):
                    rvec = jnp.full((LANES,), r, jnp.int32)
                    vals = ibuf[r, pl.ds(g * LANES, LANES)]
                    plsc.store_scatter(obuf, [rvec, pos], vals)
            return pass_fn

        plsc.parallel_loop(0, G, 1, unroll=4)(make_pass(ibuf, obuf))

        cp = pltpu.make_async_copy(
            obuf,
            out_hbm.at[pl.ds(r0 + k * RPP, RPP), pl.ds(col0, L)],
            out_sems[slot])
        cp.start()
        out_pending[slot] = cp

    for slot in range(2):
        if out_pending[slot] is not None:
            out_pending[slot].wait()


@functools.partial(jax.jit, static_argnames=())
def _run(p, species):
    mesh = plsc.VectorSubcoreMesh(core_axis_name="c", subcore_axis_name="s",
                                  num_cores=NC, num_subcores=NS)
    f = pl.kernel(
        _body,
        out_type=jax.ShapeDtypeStruct((D, N), jnp.float32),
        mesh=mesh,
        compiler_params=pltpu.CompilerParams(
            needs_layout_passes=False, use_tc_tiling_on_sc=True),
        scratch_types=[
            pltpu.VMEM((L,), jnp.int32),
            pltpu.VMEM((L,), jnp.int32),
            pltpu.VMEM((RPP, L), jnp.float32),
            pltpu.VMEM((RPP, L), jnp.float32),
            pltpu.VMEM((RPP, L), jnp.float32),
            pltpu.VMEM((RPP, L), jnp.float32),
            pltpu.VMEM((2 * LANES,), jnp.int32),
            pltpu.VMEM((2 * LANES,), jnp.int32),
            pltpu.SemaphoreType.DMA,
            pltpu.SemaphoreType.DMA,
            pltpu.SemaphoreType.DMA,
            pltpu.SemaphoreType.DMA,
        ],
    )
    return f(p, species)


def kernel(p, species, coordinates):
    del coordinates
    return _run(p, species)


# speedup vs baseline: 1.0860x; 1.0860x over previous
import functools

import jax
import jax.numpy as jnp
from jax import lax
from jax.experimental import pallas as pl
from jax.experimental.pallas import tpu as pltpu
from jax.experimental.pallas import tpu_sc as plsc

D = 64
B = 16
L = 2048
N = B * L
NC = 2
NS = 16
LANES = 16
ROWS = D // NC
G = L // LANES
RPP = 8
NPASS = ROWS // RPP


def _body(p_hbm, species_hbm, out_hbm,
          species_v, rank_v, in_b0, in_b1, in_b2, out_b0, out_b1, cnt_v,
          run_v, sem_i0, sem_i1, sem_i2, sem_o0, sem_o1):
    c = lax.axis_index("c")
    s = lax.axis_index("s")
    r0 = c * ROWS
    col0 = s * L

    in_bufs = (in_b0, in_b1, in_b2)
    out_bufs = (out_b0, out_b1)
    in_sems = (sem_i0, sem_i1, sem_i2)
    out_sems = (sem_o0, sem_o1)

    def start_in(k):
        cp = pltpu.make_async_copy(
            p_hbm.at[pl.ds(r0 + k * RPP, RPP), pl.ds(col0, L)],
            in_bufs[k % 3], in_sems[k % 3])
        cp.start()
        return cp

    in_pending = [start_in(0), start_in(1), start_in(2)]

    pltpu.sync_copy(species_hbm.at[s], species_v)

    iota = lax.iota(jnp.int32, LANES)
    zeros = jnp.zeros((LANES,), jnp.int32)
    cnt_v[pl.ds(0, LANES)] = zeros

    def cnt_body(g, _):
        svec = species_v[pl.ds(g * LANES, LANES)]
        sc, last = plsc.scan_count(svec)
        tgt = jnp.where(last, svec, LANES + iota)
        plsc.addupdate_scatter(cnt_v, [tgt], sc)
        return 0

    lax.fori_loop(0, G, cnt_body, 0)

    cnt = cnt_v[pl.ds(0, LANES)]
    off = L - plsc.cumsum(cnt)
    run_v[pl.ds(0, LANES)] = off

    def rank_body(g, _):
        svec = species_v[pl.ds(g * LANES, LANES)]
        sc, last = plsc.scan_count(svec)
        base = plsc.load_gather(run_v, [svec])
        rank_v[pl.ds(g * LANES, LANES)] = jnp.clip(base + sc - 1, 0, L - 1)
        tgt = jnp.where(last, svec, LANES + iota)
        plsc.addupdate_scatter(run_v, [tgt], sc)
        return 0

    lax.fori_loop(0, G, rank_body, 0)

    out_pending = [None, None]
    for k in range(NPASS):
        oslot = k % 2
        in_pending[k].wait()
        if k + 3 < NPASS:
            in_pending.append(start_in(k + 3))
        if out_pending[oslot] is not None:
            out_pending[oslot].wait()
        ibuf = in_bufs[k % 3]
        obuf = out_bufs[oslot]

        def make_pass(ibuf, obuf):
            def pass_fn(g):
                pos = rank_v[pl.ds(g * LANES, LANES)]
                for r in range(RPP):
                    rvec = jnp.full((LANES,), r, jnp.int32)
                    vals = ibuf[r, pl.ds(g * LANES, LANES)]
                    plsc.store_scatter(obuf, [rvec, pos], vals)
            return pass_fn

        plsc.parallel_loop(0, G, 1, unroll=2)(make_pass(ibuf, obuf))

        cp = pltpu.make_async_copy(
            obuf,
            out_hbm.at[pl.ds(r0 + k * RPP, RPP), pl.ds(col0, L)],
            out_sems[oslot])
        cp.start()
        out_pending[oslot] = cp

    for slot in range(2):
        if out_pending[slot] is not None:
            out_pending[slot].wait()


@functools.partial(jax.jit, static_argnames=())
def _run(p, species):
    mesh = plsc.VectorSubcoreMesh(core_axis_name="c", subcore_axis_name="s",
                                  num_cores=NC, num_subcores=NS)
    f = pl.kernel(
        _body,
        out_type=jax.ShapeDtypeStruct((D, N), jnp.float32),
        mesh=mesh,
        compiler_params=pltpu.CompilerParams(
            needs_layout_passes=False, use_tc_tiling_on_sc=True),
        scratch_types=[
            pltpu.VMEM((L,), jnp.int32),
            pltpu.VMEM((L,), jnp.int32),
            pltpu.VMEM((RPP, L), jnp.float32),
            pltpu.VMEM((RPP, L), jnp.float32),
            pltpu.VMEM((RPP, L), jnp.float32),
            pltpu.VMEM((RPP, L), jnp.float32),
            pltpu.VMEM((RPP, L), jnp.float32),
            pltpu.VMEM((2 * LANES,), jnp.int32),
            pltpu.VMEM((2 * LANES,), jnp.int32),
            pltpu.SemaphoreType.DMA,
            pltpu.SemaphoreType.DMA,
            pltpu.SemaphoreType.DMA,
            pltpu.SemaphoreType.DMA,
            pltpu.SemaphoreType.DMA,
        ],
    )
    return f(p, species)


def kernel(p, species, coordinates):
    del coordinates
    return _run(p, species)


# speedup vs baseline: 1.1818x; 1.0882x over previous
import functools

import jax
import jax.numpy as jnp
from jax import lax
from jax.experimental import pallas as pl
from jax.experimental.pallas import tpu as pltpu
from jax.experimental.pallas import tpu_sc as plsc

D = 64
B = 16
L = 2048
N = B * L
NC = 2
NS = 16
LANES = 16
ROWS = D // NC
G = L // LANES
RPP = 8
NPASS = ROWS // RPP


def _body(p_hbm, species_hbm, out_hbm,
          species_v, rank_v, in_b0, in_b1, in_b2, out_b0, out_b1, gh_v,
          loc_v, pre_v, run_v, sem_i0, sem_i1, sem_i2, sem_o0, sem_o1):
    c = lax.axis_index("c")
    s = lax.axis_index("s")
    r0 = c * ROWS
    col0 = s * L

    in_bufs = (in_b0, in_b1, in_b2)
    out_bufs = (out_b0, out_b1)
    in_sems = (sem_i0, sem_i1, sem_i2)
    out_sems = (sem_o0, sem_o1)

    def start_in(k):
        cp = pltpu.make_async_copy(
            p_hbm.at[pl.ds(r0 + k * RPP, RPP), pl.ds(col0, L)],
            in_bufs[k % 3], in_sems[k % 3])
        cp.start()
        return cp

    in_pending = [start_in(0), start_in(1), start_in(2)]

    pltpu.sync_copy(species_hbm.at[s], species_v)

    iota = lax.iota(jnp.int32, LANES)
    zeros = jnp.zeros((LANES,), jnp.int32)

    def zero_body(g):
        gh_v[pl.ds(g * LANES, LANES)] = zeros

    plsc.parallel_loop(0, G, 1, unroll=4)(zero_body)

    def ph1(g):
        svec = species_v[pl.ds(g * LANES, LANES)]
        sc, last = plsc.scan_count(svec)
        loc_v[pl.ds(g * LANES, LANES)] = sc - 1
        tgt = jnp.where(last, g * LANES + svec, G * LANES + iota)
        plsc.store_scatter(gh_v, [tgt], sc)

    plsc.parallel_loop(0, G, 1, unroll=2)(ph1)

    def ph2(g, run):
        pre_v[pl.ds(g * LANES, LANES)] = run
        return run + gh_v[pl.ds(g * LANES, LANES)]

    total = lax.fori_loop(0, G, ph2, zeros)

    run_v[pl.ds(0, LANES)] = L - plsc.cumsum(total)

    def ph3(g):
        svec = species_v[pl.ds(g * LANES, LANES)]
        base = plsc.load_gather(pre_v, [g * LANES + svec])
        offs = plsc.load_gather(run_v, [svec])
        pos = offs + base + loc_v[pl.ds(g * LANES, LANES)]
        rank_v[pl.ds(g * LANES, LANES)] = jnp.clip(pos, 0, L - 1)

    plsc.parallel_loop(0, G, 1, unroll=2)(ph3)

    out_pending = [None, None]
    for k in range(NPASS):
        oslot = k % 2
        in_pending[k].wait()
        if k + 3 < NPASS:
            in_pending.append(start_in(k + 3))
        if out_pending[oslot] is not None:
            out_pending[oslot].wait()
        ibuf = in_bufs[k % 3]
        obuf = out_bufs[oslot]

        def make_pass(ibuf, obuf):
            def pass_fn(g):
                pos = rank_v[pl.ds(g * LANES, LANES)]
                for r in range(RPP):
                    rvec = jnp.full((LANES,), r, jnp.int32)
                    vals = ibuf[r, pl.ds(g * LANES, LANES)]
                    plsc.store_scatter(obuf, [rvec, pos], vals)
            return pass_fn

        plsc.parallel_loop(0, G, 1, unroll=2)(make_pass(ibuf, obuf))

        cp = pltpu.make_async_copy(
            obuf,
            out_hbm.at[pl.ds(r0 + k * RPP, RPP), pl.ds(col0, L)],
            out_sems[oslot])
        cp.start()
        out_pending[oslot] = cp

    for slot in range(2):
        if out_pending[slot] is not None:
            out_pending[slot].wait()


@functools.partial(jax.jit, static_argnames=())
def _run(p, species):
    mesh = plsc.VectorSubcoreMesh(core_axis_name="c", subcore_axis_name="s",
                                  num_cores=NC, num_subcores=NS)
    f = pl.kernel(
        _body,
        out_type=jax.ShapeDtypeStruct((D, N), jnp.float32),
        mesh=mesh,
        compiler_params=pltpu.CompilerParams(
            needs_layout_passes=False, use_tc_tiling_on_sc=True),
        scratch_types=[
            pltpu.VMEM((L,), jnp.int32),
            pltpu.VMEM((L,), jnp.int32),
            pltpu.VMEM((RPP, L), jnp.float32),
            pltpu.VMEM((RPP, L), jnp.float32),
            pltpu.VMEM((RPP, L), jnp.float32),
            pltpu.VMEM((RPP, L), jnp.float32),
            pltpu.VMEM((RPP, L), jnp.float32),
            pltpu.VMEM((G * LANES + LANES,), jnp.int32),
            pltpu.VMEM((L,), jnp.int32),
            pltpu.VMEM((G * LANES,), jnp.int32),
            pltpu.VMEM((2 * LANES,), jnp.int32),
            pltpu.SemaphoreType.DMA,
            pltpu.SemaphoreType.DMA,
            pltpu.SemaphoreType.DMA,
            pltpu.SemaphoreType.DMA,
            pltpu.SemaphoreType.DMA,
        ],
    )
    return f(p, species)


def kernel(p, species, coordinates):
    del coordinates
    return _run(p, species)
